# baseline (speedup 1.0000x reference)
import jax
import jax.numpy as jnp
from jax import lax
from jax.experimental import pallas as pl
from jax.experimental.pallas import tpu as pltpu
from jax.experimental.pallas import tpu_sc as plsc

N = 2048
D = 512
E = 65536
P = 8
NC, NS = 2, 16
NW = NC * NS

CH = 128
EPW = E // NW
NCHUNK = EPW // CH

EPS = E // NS
BLK = 64
NBLK = N // BLK
BPC = NBLK // NC
ACC_ROWS = BLK * N
DUMMY_ROWS = 512
ROWS_PER_SUB = BLK // NS
MYROWS = ROWS_PER_SUB * N
ZCH = 2048

EB = 512
RB = 16

_mesh = plsc.VectorSubcoreMesh(
    core_axis_name="c", subcore_axis_name="s", num_cores=NC, num_subcores=NS)
_sc_params = pltpu.CompilerParams(use_tc_tiling_on_sc=False)


def _proj_body(obj_ref, wst_ref, wot_ref, bs_ref, bo_ref, sd_ref,
               s_ref, o_ref, head_ref):
    obj = obj_ref[...]
    s_ref[...] = (jnp.dot(obj, wst_ref[...], preferred_element_type=jnp.float32)
                  + bs_ref[0:1, :])
    o_ref[...] = (jnp.dot(obj, wot_ref[...], preferred_element_type=jnp.float32)
                  + bo_ref[0:1, :])
    head_ref[...] = jnp.full((8, 128), jnp.min(sd_ref[...]), jnp.int32)


_proj_call = pl.pallas_call(
    _proj_body,
    out_shape=[
        jax.ShapeDtypeStruct((N, D), jnp.float32),
        jax.ShapeDtypeStruct((N, D), jnp.float32),
        jax.ShapeDtypeStruct((8, 128), jnp.int32),
    ],
)


def _gather_body(sproj_h, oproj_h, src_h, dst_h, sg_h, og_h,
                 idx_v, rows_v, sem):
    cid = lax.axis_index("c")
    sid = lax.axis_index("s")
    wid = sid * NC + cid
    base = wid * EPW

    def chunk(i, carry):
        e0 = pl.multiple_of(base + i * CH, CH)
        pltpu.sync_copy(src_h.at[pl.ds(e0, CH)], idx_v)
        pltpu.async_copy(sproj_h.at[idx_v], rows_v, sem).wait()
        pltpu.sync_copy(rows_v, sg_h.at[pl.ds(e0, CH)])
        pltpu.sync_copy(dst_h.at[pl.ds(e0, CH)], idx_v)
        pltpu.async_copy(oproj_h.at[idx_v], rows_v, sem).wait()
        pltpu.sync_copy(rows_v, og_h.at[pl.ds(e0, CH)])
        return carry

    lax.fori_loop(0, NCHUNK, chunk, 0)


_gather_call = pl.kernel(
    _gather_body,
    out_type=[
        jax.ShapeDtypeStruct((E, D), jnp.float32),
        jax.ShapeDtypeStruct((E, D), jnp.float32),
    ],
    mesh=_mesh,
    compiler_params=_sc_params,
    scratch_types=[
        pltpu.VMEM((CH,), jnp.int32),
        pltpu.VMEM((CH, D), jnp.float32),
        pltpu.SemaphoreType.DMA,
    ],
)


def _af_body(sg_ref, og_ref, u_ref, ww_ref, bw_ref, af_ref):
    prod = sg_ref[...] * og_ref[...] * u_ref[...]
    af_ref[...] = (
        lax.dot_general(ww_ref[...], prod, (((1,), (1,)), ((), ())),
                        preferred_element_type=jnp.float32)
        + bw_ref[:, 0:1])


_af_call = pl.pallas_call(
    _af_body,
    grid=(E // EB,),
    in_specs=[
        pl.BlockSpec((EB, D), lambda i: (i, 0)),
        pl.BlockSpec((EB, D), lambda i: (i, 0)),
        pl.BlockSpec((EB, D), lambda i: (i, 0)),
        pl.BlockSpec((P, D), lambda i: (0, 0)),
        pl.BlockSpec((P, 128), lambda i: (0, 0)),
    ],
    out_specs=pl.BlockSpec((P, EB), lambda i: (0, i)),
    out_shape=jax.ShapeDtypeStruct((P, E), jnp.float32),
)


def _scatter_body(af_h, src_h, dst_h, head_h, zeros_h, t_h,
                  af2_v, src_v, dst_v, cidx_v, zbuf_v, head_v, acc_ref):
    cid = lax.axis_index("c")
    sid = lax.axis_index("s")
    pltpu.sync_copy(head_h, head_v)
    hv = head_v[...]
    e0 = sid * EPS
    pltpu.sync_copy(src_h.at[pl.ds(e0, EPS)], src_v)
    pltpu.sync_copy(dst_h.at[pl.ds(e0, EPS)], dst_v)
    pltpu.sync_copy(af_h.at[pl.ds(e0, EPS)], af2_v)
    pltpu.sync_copy(zeros_h, zbuf_v)

    lane = lax.iota(jnp.int32, 16)
    myrow0 = sid * MYROWS

    acc_sh = acc_ref

    def run_block(b, carry, acc_sh):
        base = (cid * BPC + b) * BLK
        for z in range(MYROWS // ZCH):
            pltpu.sync_copy(zbuf_v, acc_sh.at[pl.ds(myrow0 + z * ZCH, ZCH)])
        plsc.subcore_barrier()

        def civ(i, c2):
            sv = src_v[pl.ds(i * 16, 16)] - hv - base
            dv = dst_v[pl.ds(i * 16, 16)] - hv
            ok = (sv >= 0) & (sv < BLK)
            ii = sv * N + dv
            dmy = ACC_ROWS + (i % 32) * 16 + lane
            cidx_v[i // 8, pl.ds((i % 8) * 16, 16)] = jnp.where(ok, ii, dmy)
            return c2

        lax.fori_loop(0, EPS // 16, civ, 0)
        for j in range(EPS // CH):
            pltpu.sync_copy(af2_v.at[pl.ds(j * CH, CH)],
                            acc_sh.at[cidx_v.at[j]], add=True)
        plsc.subcore_barrier()
        r0 = base + sid * ROWS_PER_SUB
        for z in range(ROWS_PER_SUB):
            pltpu.sync_copy(acc_sh.at[pl.ds(myrow0 + z * ZCH, ZCH)],
                            t_h.at[r0 + z])
        return carry

    lax.fori_loop(0, BPC, lambda b, c: run_block(b, c, acc_sh), 0)


_scatter_call = pl.kernel(
    _scatter_body,
    out_type=jax.ShapeDtypeStruct((N, N, P), jnp.float32),
    mesh=_mesh,
    compiler_params=_sc_params,
    scratch_types=[
        pltpu.VMEM((EPS, P), jnp.float32),
        pltpu.VMEM((EPS,), jnp.int32),
        pltpu.VMEM((EPS,), jnp.int32),
        pltpu.VMEM((EPS // CH, CH), jnp.int32),
        pltpu.VMEM((ZCH, P), jnp.float32),
        pltpu.VMEM((16,), jnp.int32),
        pltpu.VMEM_SHARED((ACC_ROWS + DUMMY_ROWS, P), jnp.float32),
    ],
)


def _rowsum_body(t_ref, bias_ref, s_ref):
    b = pl.program_id(0)
    a = jax.nn.sigmoid(t_ref[...].reshape(RB, N, P) + bias_ref[0, 0])
    ri = b * RB + lax.broadcasted_iota(jnp.int32, (RB, N, P), 0)
    ci = lax.broadcasted_iota(jnp.int32, (RB, N, P), 1)
    a = jnp.where(ri == ci, 0.0, a)
    s_ref[...] = 1.0 / jnp.sum(a, axis=1)


_rowsum_call = pl.pallas_call(
    _rowsum_body,
    grid=(N // RB,),
    in_specs=[
        pl.BlockSpec((RB, N * P), lambda i: (i, 0)),
        pl.BlockSpec((8, 128), lambda i: (0, 0)),
    ],
    out_specs=pl.BlockSpec((RB, P), lambda i: (i, 0)),
    out_shape=jax.ShapeDtypeStruct((N, P), jnp.float32),
)


def _norm_body(t_ref, s_ref, bias_ref, o_ref):
    b = pl.program_id(0)
    a = jax.nn.sigmoid(t_ref[...].reshape(RB, N, P) + bias_ref[0, 0])
    ri = b * RB + lax.broadcasted_iota(jnp.int32, (RB, N, P), 0)
    ci = lax.broadcasted_iota(jnp.int32, (RB, N, P), 1)
    a = jnp.where(ri == ci, 0.0, a)
    o_ref[...] = (a * s_ref[...][None, :, :]).reshape(RB, N * P)


_norm_call = pl.pallas_call(
    _norm_body,
    grid=(N // RB,),
    in_specs=[
        pl.BlockSpec((RB, N * P), lambda i: (i, 0)),
        pl.BlockSpec((N, P), lambda i: (0, 0)),
        pl.BlockSpec((8, 128), lambda i: (0, 0)),
    ],
    out_specs=pl.BlockSpec((RB, N * P), lambda i: (i, 0)),
    out_shape=jax.ShapeDtypeStruct((N, N * P), jnp.float32),
)


def kernel(obj_feats, rel_inds, union_feats, n_nodes, Ws, bs, Wo, bo, Ww, bw):
    src = rel_inds[:, 1]
    dst = rel_inds[:, 2]
    sd = rel_inds[:, 1:3].reshape(E // 64, 128)
    bias = jnp.broadcast_to(
        (jnp.asarray(n_nodes, jnp.float32) - jnp.float32(N)).reshape(1, 1),
        (8, 128))

    s_proj, o_proj, head_arr = _proj_call(
        obj_feats, Ws.T, Wo.T,
        jnp.broadcast_to(bs.reshape(1, D), (8, D)),
        jnp.broadcast_to(bo.reshape(1, D), (8, D)), sd)
    head16 = head_arr[0, :16]

    sg, og = _gather_call(s_proj, o_proj, src, dst)
    af_t = _af_call(sg, og, union_feats, Ww,
                    jnp.broadcast_to(bw.reshape(P, 1), (P, 128)))

    t = _scatter_call(af_t.T, src, dst, head16,
                      jnp.zeros((ZCH, P), jnp.float32))
    t2 = t.reshape(N, N * P)
    s_sums = _rowsum_call(t2, bias)
    out = _norm_call(t2, s_sums, bias)
    return out.reshape(N, N, P)


# speedup vs baseline: 2.0965x; 2.0965x over previous
import jax
import jax.numpy as jnp
from jax import lax
from jax.experimental import pallas as pl
from jax.experimental.pallas import tpu as pltpu
from jax.experimental.pallas import tpu_sc as plsc

N = 2048
D = 512
E = 65536
P = 8
NC, NS = 2, 16
NW = NC * NS

CH = 128
EPW = E // NW
NCHUNK = EPW // CH

EPS = E // NS
BLK = 64
NBLK = N // BLK
BPC = NBLK // NC
ACC_ROWS = BLK * N
DUMMY_ROWS = 512
ROWS_PER_SUB = BLK // NS
MYROWS = ROWS_PER_SUB * N
ZCH = 2048

EB = 512
RB = 16

_mesh = plsc.VectorSubcoreMesh(
    core_axis_name="c", subcore_axis_name="s", num_cores=NC, num_subcores=NS)
_sc_params = pltpu.CompilerParams(use_tc_tiling_on_sc=False)


def _proj_body(obj_ref, wst_ref, wot_ref, bs_ref, bo_ref, sd_ref,
               s_ref, o_ref, head_ref):
    obj = obj_ref[...]
    s_ref[...] = (jnp.dot(obj, wst_ref[...], preferred_element_type=jnp.float32)
                  + bs_ref[0:1, :])
    o_ref[...] = (jnp.dot(obj, wot_ref[...], preferred_element_type=jnp.float32)
                  + bo_ref[0:1, :])
    head_ref[...] = jnp.full((8, 128), jnp.min(sd_ref[...]), jnp.int32)


_proj_call = pl.pallas_call(
    _proj_body,
    out_shape=[
        jax.ShapeDtypeStruct((N, D), jnp.float32),
        jax.ShapeDtypeStruct((N, D), jnp.float32),
        jax.ShapeDtypeStruct((8, 128), jnp.int32),
    ],
)


def _gather_body(sproj_h, oproj_h, src_h, dst_h, sg_h, og_h,
                 idx_v, rows_v, sem):
    cid = lax.axis_index("c")
    sid = lax.axis_index("s")
    wid = sid * NC + cid
    base = wid * EPW

    def chunk(i, carry):
        e0 = pl.multiple_of(base + i * CH, CH)
        pltpu.sync_copy(src_h.at[pl.ds(e0, CH)], idx_v)
        pltpu.async_copy(sproj_h.at[idx_v], rows_v, sem).wait()
        pltpu.sync_copy(rows_v, sg_h.at[pl.ds(e0, CH)])
        pltpu.sync_copy(dst_h.at[pl.ds(e0, CH)], idx_v)
        pltpu.async_copy(oproj_h.at[idx_v], rows_v, sem).wait()
        pltpu.sync_copy(rows_v, og_h.at[pl.ds(e0, CH)])
        return carry

    lax.fori_loop(0, NCHUNK, chunk, 0)


_gather_call = pl.kernel(
    _gather_body,
    out_type=[
        jax.ShapeDtypeStruct((E, D), jnp.float32),
        jax.ShapeDtypeStruct((E, D), jnp.float32),
    ],
    mesh=_mesh,
    compiler_params=_sc_params,
    scratch_types=[
        pltpu.VMEM((CH,), jnp.int32),
        pltpu.VMEM((CH, D), jnp.float32),
        pltpu.SemaphoreType.DMA,
    ],
)


def _af_body(sg_ref, og_ref, u_ref, ww_ref, bw_ref, af_ref):
    prod = sg_ref[...] * og_ref[...] * u_ref[...]
    af_ref[...] = (
        lax.dot_general(ww_ref[...], prod, (((1,), (1,)), ((), ())),
                        preferred_element_type=jnp.float32)
        + bw_ref[:, 0:1])


_af_call = pl.pallas_call(
    _af_body,
    grid=(E // EB,),
    in_specs=[
        pl.BlockSpec((EB, D), lambda i: (i, 0)),
        pl.BlockSpec((EB, D), lambda i: (i, 0)),
        pl.BlockSpec((EB, D), lambda i: (i, 0)),
        pl.BlockSpec((P, D), lambda i: (0, 0)),
        pl.BlockSpec((P, 128), lambda i: (0, 0)),
    ],
    out_specs=pl.BlockSpec((P, EB), lambda i: (0, i)),
    out_shape=jax.ShapeDtypeStruct((P, E), jnp.float32),
)


def _scatter_body(af_h, src_h, dst_h, head_h, zeros_h, t_h,
                  af2_v, src_v, dst_v, cidx_v, zbuf_v, head_v, acc_ref):
    cid = lax.axis_index("c")
    sid = lax.axis_index("s")
    pltpu.sync_copy(head_h, head_v)
    hv = head_v[...]
    e0 = sid * EPS
    pltpu.sync_copy(src_h.at[pl.ds(e0, EPS)], src_v)
    pltpu.sync_copy(dst_h.at[pl.ds(e0, EPS)], dst_v)
    pltpu.sync_copy(af_h.at[pl.ds(e0, EPS)], af2_v)
    pltpu.sync_copy(zeros_h, zbuf_v)

    lane = lax.iota(jnp.int32, 16)
    myrow0 = sid * MYROWS

    acc_sh = acc_ref

    def run_block(b, carry, acc_sh):
        base = (cid * BPC + b) * BLK
        for z in range(MYROWS // ZCH):
            pltpu.sync_copy(zbuf_v, acc_sh.at[pl.ds(myrow0 + z * ZCH, ZCH)])
        plsc.subcore_barrier()

        def civ(i, c2):
            sv = src_v[pl.ds(i * 16, 16)] - hv - base
            dv = dst_v[pl.ds(i * 16, 16)] - hv
            ok = (sv >= 0) & (sv < BLK)
            ii = sv * N + dv
            dmy = ACC_ROWS + (i % 32) * 16 + lane
            cidx_v[i // 8, pl.ds((i % 8) * 16, 16)] = jnp.where(ok, ii, dmy)
            return c2

        lax.fori_loop(0, EPS // 16, civ, 0)
        for j in range(EPS // CH):
            pltpu.sync_copy(af2_v.at[pl.ds(j * CH, CH)],
                            acc_sh.at[cidx_v.at[j]], add=True)
        plsc.subcore_barrier()
        r0 = base + sid * ROWS_PER_SUB
        for z in range(ROWS_PER_SUB):
            pltpu.sync_copy(acc_sh.at[pl.ds(myrow0 + z * ZCH, ZCH)],
                            t_h.at[r0 + z])
        return carry

    lax.fori_loop(0, BPC, lambda b, c: run_block(b, c, acc_sh), 0)


_scatter_call = pl.kernel(
    _scatter_body,
    out_type=jax.ShapeDtypeStruct((N, N, P), jnp.float32),
    mesh=_mesh,
    compiler_params=_sc_params,
    scratch_types=[
        pltpu.VMEM((EPS, P), jnp.float32),
        pltpu.VMEM((EPS,), jnp.int32),
        pltpu.VMEM((EPS,), jnp.int32),
        pltpu.VMEM((EPS // CH, CH), jnp.int32),
        pltpu.VMEM((ZCH, P), jnp.float32),
        pltpu.VMEM((16,), jnp.int32),
        pltpu.VMEM_SHARED((ACC_ROWS + DUMMY_ROWS, P), jnp.float32),
    ],
)


TROWS = N * N * P // 128
BR = RB * N * P // 128


def _sig_mask_block(t_ref, bias_ref, b):
    a = jax.nn.sigmoid(t_ref[...] + bias_ref[0, 0])
    ri = lax.broadcasted_iota(jnp.int32, (BR, 128), 0)
    li = lax.broadcasted_iota(jnp.int32, (BR, 128), 1)
    m = (ri % 128) * 16 + li // P
    i_glob = b * RB + ri // 128
    return jnp.where(m == i_glob, 0.0, a)


def _rowsum_body(t_ref, bias_ref, s_ref):
    b = pl.program_id(0)
    a = _sig_mask_block(t_ref, bias_ref, b)
    ssum = jnp.sum(a.reshape(RB, 128, 128), axis=1)
    i0 = lax.broadcasted_iota(jnp.int32, (128, 128), 0)
    i1 = lax.broadcasted_iota(jnp.int32, (128, 128), 1)
    m2 = (i0 % P == i1 % P).astype(jnp.float32)
    q = jnp.dot(ssum, m2, preferred_element_type=jnp.float32)
    w0 = lax.broadcasted_iota(jnp.int32, (RB, 128), 0)
    w1 = lax.broadcasted_iota(jnp.int32, (RB, 128), 1)
    sel = (w0 == w1 // P).astype(jnp.float32)
    s_ref[...] = (1.0 / jnp.sum(q * sel, axis=0, keepdims=True)).reshape(1, 1, 128)


_rowsum_call = pl.pallas_call(
    _rowsum_body,
    grid=(N // RB,),
    in_specs=[
        pl.BlockSpec((BR, 128), lambda i: (i, 0)),
        pl.BlockSpec((8, 128), lambda i: (0, 0)),
    ],
    out_specs=pl.BlockSpec((1, 1, 128), lambda i: (i, 0, 0)),
    out_shape=jax.ShapeDtypeStruct((N // RB, 1, 128), jnp.float32),
)


def _norm_body(t_ref, s_ref, bias_ref, o_ref):
    b = pl.program_id(0)
    a = _sig_mask_block(t_ref, bias_ref, b)
    rs = s_ref[...].reshape(128, 128)
    factor = jnp.broadcast_to(rs[None], (RB, 128, 128)).reshape(BR, 128)
    o_ref[...] = a * factor


_norm_call = pl.pallas_call(
    _norm_body,
    grid=(N // RB,),
    in_specs=[
        pl.BlockSpec((BR, 128), lambda i: (i, 0)),
        pl.BlockSpec((N // RB, 1, 128), lambda i: (0, 0, 0)),
        pl.BlockSpec((8, 128), lambda i: (0, 0)),
    ],
    out_specs=pl.BlockSpec((BR, 128), lambda i: (i, 0)),
    out_shape=jax.ShapeDtypeStruct((TROWS, 128), jnp.float32),
)


def kernel(obj_feats, rel_inds, union_feats, n_nodes, Ws, bs, Wo, bo, Ww, bw):
    src = rel_inds[:, 1]
    dst = rel_inds[:, 2]
    sd = rel_inds[:, 1:3].reshape(E // 64, 128)
    bias = jnp.broadcast_to(
        (jnp.asarray(n_nodes, jnp.float32) - jnp.float32(N)).reshape(1, 1),
        (8, 128))

    s_proj, o_proj, head_arr = _proj_call(
        obj_feats, Ws.T, Wo.T,
        jnp.broadcast_to(bs.reshape(1, D), (8, D)),
        jnp.broadcast_to(bo.reshape(1, D), (8, D)), sd)
    head16 = head_arr[0, :16]

    sg, og = _gather_call(s_proj, o_proj, src, dst)
    af_t = _af_call(sg, og, union_feats, Ww,
                    jnp.broadcast_to(bw.reshape(P, 1), (P, 128)))

    t = _scatter_call(af_t.T, src, dst, head16,
                      jnp.zeros((ZCH, P), jnp.float32))
    t128 = t.reshape(TROWS, 128)
    s_recip = _rowsum_call(t128, bias)
    out = _norm_call(t128, s_recip, bias)
    return out.reshape(N, N, P)


# speedup vs baseline: 2.2983x; 1.0963x over previous
import jax
import jax.numpy as jnp
from jax import lax
from jax.experimental import pallas as pl
from jax.experimental.pallas import tpu as pltpu
from jax.experimental.pallas import tpu_sc as plsc

N = 2048
D = 512
E = 65536
P = 8
NC, NS = 2, 16
NW = NC * NS

CH = 128
EPW = E // NW
NCHUNK = EPW // CH

EPS = E // NS
BLK = 64
NBLK = N // BLK
BPC = NBLK // NC
ACC_ROWS = BLK * N
DUMMY_ROWS = 512
ROWS_PER_SUB = BLK // NS
MYROWS = ROWS_PER_SUB * N
ZCH = 2048

RB = 16

_mesh = plsc.VectorSubcoreMesh(
    core_axis_name="c", subcore_axis_name="s", num_cores=NC, num_subcores=NS)
_sc_params = pltpu.CompilerParams(use_tc_tiling_on_sc=False)


def _proj_body(obj_ref, wst_ref, wot_ref, bs_ref, bo_ref, sd_ref,
               s_ref, o_ref, head_ref):
    obj = obj_ref[...]
    s_ref[...] = (jnp.dot(obj, wst_ref[...], preferred_element_type=jnp.float32)
                  + bs_ref[0:1, :])
    o_ref[...] = (jnp.dot(obj, wot_ref[...], preferred_element_type=jnp.float32)
                  + bo_ref[0:1, :])
    head_ref[...] = jnp.full((8, 128), jnp.min(sd_ref[...]), jnp.int32)


_proj_call = pl.pallas_call(
    _proj_body,
    out_shape=[
        jax.ShapeDtypeStruct((N, D), jnp.float32),
        jax.ShapeDtypeStruct((N, D), jnp.float32),
        jax.ShapeDtypeStruct((8, 128), jnp.int32),
    ],
)


def _gather_body(sproj_h, oproj_h, src_h, dst_h, sg_h, og_h,
                 idx_v, rows_v, sem):
    cid = lax.axis_index("c")
    sid = lax.axis_index("s")
    wid = sid * NC + cid
    base = wid * EPW

    def chunk(i, carry):
        e0 = pl.multiple_of(base + i * CH, CH)
        pltpu.sync_copy(src_h.at[pl.ds(e0, CH)], idx_v)
        pltpu.async_copy(sproj_h.at[idx_v], rows_v, sem).wait()
        pltpu.sync_copy(rows_v, sg_h.at[pl.ds(e0, CH)])
        pltpu.sync_copy(dst_h.at[pl.ds(e0, CH)], idx_v)
        pltpu.async_copy(oproj_h.at[idx_v], rows_v, sem).wait()
        pltpu.sync_copy(rows_v, og_h.at[pl.ds(e0, CH)])
        return carry

    lax.fori_loop(0, NCHUNK, chunk, 0)


_gather_call = pl.kernel(
    _gather_body,
    out_type=[
        jax.ShapeDtypeStruct((E, D), jnp.float32),
        jax.ShapeDtypeStruct((E, D), jnp.float32),
    ],
    mesh=_mesh,
    compiler_params=_sc_params,
    scratch_types=[
        pltpu.VMEM((CH,), jnp.int32),
        pltpu.VMEM((CH, D), jnp.float32),
        pltpu.SemaphoreType.DMA,
    ],
)


EB = 1024
VR = 4 * EB


def _af_body(sg_ref, og_ref, u_ref, ww_ref, bw_ref, af_ref):
    p = (sg_ref[...] * og_ref[...] * u_ref[...]).reshape(EB, 4, 128)
    x = sum(
        jnp.dot(p[:, j, :], ww_ref[j], preferred_element_type=jnp.float32)
        for j in range(4))
    x3 = x.reshape(EB // 16, 16, 128)
    k2d = (lax.broadcasted_iota(jnp.int32, (16, 128), 0)
           == lax.broadcasted_iota(jnp.int32, (16, 128), 1) // P
           ).astype(jnp.float32)
    af_ref[...] = jnp.sum(x3 * k2d[None], axis=1) + bw_ref[0:1, :]


_af_call = pl.pallas_call(
    _af_body,
    grid=(E // EB,),
    in_specs=[
        pl.BlockSpec((VR, 128), lambda i: (i, 0)),
        pl.BlockSpec((VR, 128), lambda i: (i, 0)),
        pl.BlockSpec((VR, 128), lambda i: (i, 0)),
        pl.BlockSpec((4, 128, 128), lambda i: (0, 0, 0)),
        pl.BlockSpec((8, 128), lambda i: (0, 0)),
    ],
    out_specs=pl.BlockSpec((EB * P // 128, 128), lambda i: (i, 0)),
    out_shape=jax.ShapeDtypeStruct((E * P // 128, 128), jnp.float32),
)


def _scatter_body(af_h, src_h, dst_h, head_h, zeros_h, t_h,
                  af2_v, src_v, dst_v, cidx_v, zbuf_v, head_v, acc_ref):
    cid = lax.axis_index("c")
    sid = lax.axis_index("s")
    pltpu.sync_copy(head_h, head_v)
    hv = head_v[...]
    e0 = sid * EPS
    pltpu.sync_copy(src_h.at[pl.ds(e0, EPS)], src_v)
    pltpu.sync_copy(dst_h.at[pl.ds(e0, EPS)], dst_v)
    pltpu.sync_copy(af_h.at[pl.ds(e0, EPS)], af2_v)
    pltpu.sync_copy(zeros_h, zbuf_v)

    lane = lax.iota(jnp.int32, 16)
    myrow0 = sid * MYROWS

    acc_sh = acc_ref

    def run_block(b, carry, acc_sh):
        base = (cid * BPC + b) * BLK
        for z in range(MYROWS // ZCH):
            pltpu.sync_copy(zbuf_v, acc_sh.at[pl.ds(myrow0 + z * ZCH, ZCH)])
        plsc.subcore_barrier()

        def civ(i, c2):
            sv = src_v[pl.ds(i * 16, 16)] - hv - base
            dv = dst_v[pl.ds(i * 16, 16)] - hv
            ok = (sv >= 0) & (sv < BLK)
            ii = sv * N + dv
            dmy = ACC_ROWS + (i % 32) * 16 + lane
            cidx_v[i // 8, pl.ds((i % 8) * 16, 16)] = jnp.where(ok, ii, dmy)
            return c2

        lax.fori_loop(0, EPS // 16, civ, 0)
        for j in range(EPS // CH):
            pltpu.sync_copy(af2_v.at[pl.ds(j * CH, CH)],
                            acc_sh.at[cidx_v.at[j]], add=True)
        plsc.subcore_barrier()
        r0 = base + sid * ROWS_PER_SUB
        for z in range(ROWS_PER_SUB):
            pltpu.sync_copy(acc_sh.at[pl.ds(myrow0 + z * ZCH, ZCH)],
                            t_h.at[r0 + z])
        return carry

    lax.fori_loop(0, BPC, lambda b, c: run_block(b, c, acc_sh), 0)


_scatter_call = pl.kernel(
    _scatter_body,
    out_type=jax.ShapeDtypeStruct((N, N, P), jnp.float32),
    mesh=_mesh,
    compiler_params=_sc_params,
    scratch_types=[
        pltpu.VMEM((EPS, P), jnp.float32),
        pltpu.VMEM((EPS,), jnp.int32),
        pltpu.VMEM((EPS,), jnp.int32),
        pltpu.VMEM((EPS // CH, CH), jnp.int32),
        pltpu.VMEM((ZCH, P), jnp.float32),
        pltpu.VMEM((16,), jnp.int32),
        pltpu.VMEM_SHARED((ACC_ROWS + DUMMY_ROWS, P), jnp.float32),
    ],
)


TROWS = N * N * P // 128
BR = RB * N * P // 128


def _sig_mask_block(t_ref, bias_ref, b):
    a = jax.nn.sigmoid(t_ref[...] + bias_ref[0, 0])
    ri = lax.broadcasted_iota(jnp.int32, (BR, 128), 0)
    li = lax.broadcasted_iota(jnp.int32, (BR, 128), 1)
    m = (ri % 128) * 16 + li // P
    i_glob = b * RB + ri // 128
    return jnp.where(m == i_glob, 0.0, a)


def _rowsum_body(t_ref, bias_ref, s_ref):
    b = pl.program_id(0)
    a = _sig_mask_block(t_ref, bias_ref, b)
    ssum = jnp.sum(a.reshape(RB, 128, 128), axis=1)
    i0 = lax.broadcasted_iota(jnp.int32, (128, 128), 0)
    i1 = lax.broadcasted_iota(jnp.int32, (128, 128), 1)
    m2 = (i0 % P == i1 % P).astype(jnp.float32)
    q = jnp.dot(ssum, m2, preferred_element_type=jnp.float32)
    w0 = lax.broadcasted_iota(jnp.int32, (RB, 128), 0)
    w1 = lax.broadcasted_iota(jnp.int32, (RB, 128), 1)
    sel = (w0 == w1 // P).astype(jnp.float32)
    s_ref[...] = (1.0 / jnp.sum(q * sel, axis=0, keepdims=True)).reshape(1, 1, 128)


_rowsum_call = pl.pallas_call(
    _rowsum_body,
    grid=(N // RB,),
    in_specs=[
        pl.BlockSpec((BR, 128), lambda i: (i, 0)),
        pl.BlockSpec((8, 128), lambda i: (0, 0)),
    ],
    out_specs=pl.BlockSpec((1, 1, 128), lambda i: (i, 0, 0)),
    out_shape=jax.ShapeDtypeStruct((N // RB, 1, 128), jnp.float32),
)


def _norm_body(t_ref, s_ref, bias_ref, o_ref):
    b = pl.program_id(0)
    a = _sig_mask_block(t_ref, bias_ref, b)
    rs = s_ref[...].reshape(128, 128)
    factor = jnp.broadcast_to(rs[None], (RB, 128, 128)).reshape(BR, 128)
    o_ref[...] = a * factor


_norm_call = pl.pallas_call(
    _norm_body,
    grid=(N // RB,),
    in_specs=[
        pl.BlockSpec((BR, 128), lambda i: (i, 0)),
        pl.BlockSpec((N // RB, 1, 128), lambda i: (0, 0, 0)),
        pl.BlockSpec((8, 128), lambda i: (0, 0)),
    ],
    out_specs=pl.BlockSpec((BR, 128), lambda i: (i, 0)),
    out_shape=jax.ShapeDtypeStruct((TROWS, 128), jnp.float32),
)


def kernel(obj_feats, rel_inds, union_feats, n_nodes, Ws, bs, Wo, bo, Ww, bw):
    src = rel_inds[:, 1]
    dst = rel_inds[:, 2]
    sd = rel_inds[:, 1:3].reshape(E // 64, 128)
    bias = jnp.broadcast_to(
        (jnp.asarray(n_nodes, jnp.float32) - jnp.float32(N)).reshape(1, 1),
        (8, 128))

    s_proj, o_proj, head_arr = _proj_call(
        obj_feats, Ws.T, Wo.T,
        jnp.broadcast_to(bs.reshape(1, D), (8, D)),
        jnp.broadcast_to(bo.reshape(1, D), (8, D)), sd)
    head16 = head_arr[0, :16]

    sg, og = _gather_call(s_proj, o_proj, src, dst)
    wexp = jnp.tile(Ww, (16, 1)).T.reshape(4, 128, 128)
    af_packed = _af_call(
        sg.reshape(E * 4, 128), og.reshape(E * 4, 128),
        union_feats.reshape(E * 4, 128), wexp,
        jnp.broadcast_to(jnp.tile(bw, 16)[None, :], (8, 128)))

    t = _scatter_call(af_packed.reshape(E, P), src, dst, head16,
                      jnp.zeros((ZCH, P), jnp.float32))
    t128 = t.reshape(TROWS, 128)
    s_recip = _rowsum_call(t128, bias)
    out = _norm_call(t128, s_recip, bias)
    return out.reshape(N, N, P)
